# baseline (speedup 1.0000x reference)
import jax
import jax.numpy as jnp
from jax.experimental import pallas as pl
from jax.experimental.pallas import tpu as pltpu

_DIM = 512
_E = 8
_NC = 1000
_B = 32


def _pack_conv_w(w, P):
    Cout, Cin, KH, KW = w.shape
    S = P + 2
    w_t = jnp.transpose(w, (2, 3, 1, 0))
    cols = []
    for p in range(P):
        blk = jnp.zeros((KH, S, Cin, Cout), w.dtype)
        blk = blk.at[:, p:p + 3].set(w_t)
        cols.append(blk)
    Wp = jnp.stack(cols, axis=3)
    return Wp.reshape(KH * S * Cin, P * Cout)


def _ktile(a, CL):
    H = a.shape[0] - 2
    G = a.shape[1]
    L = a.shape[2]
    z = jnp.zeros((H, 1, CL), a.dtype)
    parts = []
    for dh in range(3):
        m = a[dh:dh + H]
        left = jnp.concatenate([z, m[:, :G - 1, L - CL:]], axis=1)
        right = jnp.concatenate([m[:, 1:, :CL], z], axis=1)
        parts += [left, m, right]
    return jnp.concatenate(parts, axis=-1)


def _enc_kernel(xg_ref, w1_ref, b1_ref, w2_ref, b2_ref, w3_ref, b3_ref,
                w4_ref, b4_ref, out_ref):
    f32 = jnp.float32
    xg = xg_ref[0]
    k1 = jnp.concatenate([xg[0:224], xg[1:225], xg[2:226]], axis=-1)
    a1 = jnp.dot(k1.reshape(6272, 90), w1_ref[...],
                 preferred_element_type=f32)
    a1 = jax.nn.relu(a1 + b1_ref[...])
    act1 = a1.reshape(224, 28, 256).astype(jnp.bfloat16)
    act1 = jnp.pad(act1, ((1, 1), (0, 0), (0, 0)))
    k2 = _ktile(act1, 32)
    a2 = jnp.dot(k2.reshape(6272, 960), w2_ref[...],
                 preferred_element_type=f32)
    a2 = jax.nn.relu(a2 + b2_ref[...]).reshape(112, 2, 28, 256)
    s = a2[:, 0] + a2[:, 1]
    s4 = jnp.concatenate(
        [s[..., 0:32] + s[..., 32:64], s[..., 64:96] + s[..., 96:128],
         s[..., 128:160] + s[..., 160:192],
         s[..., 192:224] + s[..., 224:256]], axis=-1)
    act2 = (s4 * 0.25).astype(jnp.bfloat16)
    act2 = jnp.pad(act2, ((1, 1), (0, 0), (0, 0)))
    k3 = _ktile(act2, 32)
    a3 = jnp.dot(k3.reshape(3136, 576), w3_ref[...],
                 preferred_element_type=f32)
    a3 = jax.nn.relu(a3 + b3_ref[...]).reshape(56, 2, 28, 256)
    s = a3[:, 0] + a3[:, 1]
    s4 = jnp.concatenate(
        [s[..., 0:64] + s[..., 64:128], s[..., 128:192] + s[..., 192:256]],
        axis=-1)
    act3 = (s4.reshape(56, 14, 256) * 0.25).astype(jnp.bfloat16)
    act3 = jnp.pad(act3, ((1, 1), (0, 0), (0, 0)))
    k4 = _ktile(act3, 64)
    a4 = jnp.dot(k4.reshape(784, 1152), w4_ref[...],
                 preferred_element_type=f32)
    a4 = jax.nn.relu(a4 + b4_ref[...])
    r = jnp.sum(a4, axis=0, keepdims=True)
    hsum = (r[:, 0:64] + r[:, 64:128] + r[:, 128:192] + r[:, 192:256])
    out_ref[0] = hsum * (1.0 / 3136.0)


def _run_encoder(x, p):
    xb = x.astype(jnp.bfloat16)
    xp = jnp.pad(xb, ((0, 0), (0, 0), (0, 0), (1, 1)))
    slices = [xp[:, :, :, s::8][:, :, :, :28] for s in range(10)]
    xg = jnp.stack(slices, axis=-1)
    xg = jnp.transpose(xg, (0, 2, 3, 4, 1)).reshape(32, 224, 28, 30)
    xg = jnp.pad(xg, ((0, 0), (1, 1), (0, 0), (0, 0)))

    bf = jnp.bfloat16
    w1 = _pack_conv_w(p['c1w'], 8).astype(bf)
    w2 = _pack_conv_w(p['c2w'], 8).astype(bf)
    w3 = _pack_conv_w(p['c3w'], 4).astype(bf)
    w4 = _pack_conv_w(p['c4w'], 4).astype(bf)
    b1 = jnp.tile(p['c1b'], 8).reshape(1, 256)
    b2 = jnp.tile(p['c2b'], 8).reshape(1, 256)
    b3 = jnp.tile(p['c3b'], 4).reshape(1, 256)
    b4 = jnp.tile(p['c4b'], 4).reshape(1, 256)

    full = lambda shp: pl.BlockSpec(shp, lambda n: (0,) * len(shp))
    h_feat = pl.pallas_call(
        _enc_kernel,
        grid=(32,),
        in_specs=[
            pl.BlockSpec((1, 226, 28, 30), lambda n: (n, 0, 0, 0)),
            full((90, 256)), full((1, 256)),
            full((960, 256)), full((1, 256)),
            full((576, 256)), full((1, 256)),
            full((1152, 256)), full((1, 256)),
        ],
        out_specs=pl.BlockSpec((1, 1, 64), lambda n: (n, 0, 0)),
        out_shape=jax.ShapeDtypeStruct((32, 1, 64), jnp.float32),
    )(xg, w1, b1, w2, b2, w3, b3, w4, b4)
    return h_feat.reshape(32, 64)


def _head_kernel(h_ref, pw_ref, pb_ref, rw_ref, rb_ref,
                 ew1_ref, eb1_ref, ew2_ref, eb2_ref,
                 logits_ref, counts_ref, imp_ref, aux_ref,
                 z_s, oh_s):
    e = pl.program_id(0)
    f32 = jnp.float32
    bf = jnp.bfloat16

    @pl.when(e == 0)
    def _router():
        h = h_ref[...]
        z = jnp.dot(h.astype(bf), pw_ref[...].astype(bf),
                    preferred_element_type=f32) + pb_ref[...]
        gl = jnp.dot(z.astype(bf), rw_ref[...].astype(bf),
                     preferred_element_type=f32) + rb_ref[...]
        probs = jax.nn.softmax(gl, axis=-1)
        best = probs[:, 0:1]
        idx = jnp.zeros((32, 1), jnp.int32)
        for j in range(1, 8):
            cur = probs[:, j:j + 1]
            upd = cur > best
            best = jnp.where(upd, cur, best)
            idx = jnp.where(upd, jnp.int32(j), idx)
        lane = jax.lax.broadcasted_iota(jnp.int32, (32, 8), 1)
        onehot = (lane == idx).astype(f32)
        counts = jnp.sum(onehot, axis=0, keepdims=True) * (1.0 / 32.0)
        imp = jnp.sum(probs, axis=0, keepdims=True) * (1.0 / 32.0)
        counts_ref[...] = counts
        imp_ref[...] = imp
        aux_ref[...] = 8.0 * jnp.sum(counts * imp).reshape(1, 1)
        z_s[...] = z
        oh_s[...] = onehot

    z = z_s[...]
    h1 = jnp.dot(z.astype(bf), ew1_ref[0].astype(bf),
                 preferred_element_type=f32) + eb1_ref[0]
    h1 = jax.nn.relu(h1)
    o = jnp.dot(h1.astype(bf), ew2_ref[0].astype(bf),
                preferred_element_type=f32) + eb2_ref[0]
    m = jnp.sum(oh_s[...] * (jax.lax.broadcasted_iota(
        jnp.int32, (32, 8), 1) == e).astype(f32), axis=1, keepdims=True)
    val = o * m

    @pl.when(e == 0)
    def _init():
        logits_ref[...] = val

    @pl.when(e != 0)
    def _acc():
        logits_ref[...] = logits_ref[...] + val


def _run_head(h_feat, p):
    full = lambda shp: pl.BlockSpec(shp, lambda e: (0,) * len(shp))
    out_shapes = [
        jax.ShapeDtypeStruct((32, _NC), jnp.float32),
        jax.ShapeDtypeStruct((1, _E), jnp.float32),
        jax.ShapeDtypeStruct((1, _E), jnp.float32),
        jax.ShapeDtypeStruct((1, 1), jnp.float32),
    ]
    logits, counts, imp, aux = pl.pallas_call(
        _head_kernel,
        grid=(_E,),
        in_specs=[
            full((32, 64)),
            full((64, _DIM)), full((1, _DIM)),
            full((_DIM, _E)), full((1, _E)),
            pl.BlockSpec((1, _DIM, _DIM), lambda e: (e, 0, 0)),
            pl.BlockSpec((1, 1, _DIM), lambda e: (e, 0, 0)),
            pl.BlockSpec((1, _DIM, _NC), lambda e: (e, 0, 0)),
            pl.BlockSpec((1, 1, _NC), lambda e: (e, 0, 0)),
        ],
        out_specs=[
            full((32, _NC)), full((1, _E)), full((1, _E)), full((1, 1)),
        ],
        out_shape=out_shapes,
        scratch_shapes=[
            pltpu.VMEM((32, _DIM), jnp.float32),
            pltpu.VMEM((32, _E), jnp.float32),
        ],
    )(h_feat, p['pw'], p['pb'].reshape(1, _DIM), p['rw'],
      p['rb'].reshape(1, _E), p['ew1'], p['eb1'].reshape(_E, 1, _DIM),
      p['ew2'], p['eb2'].reshape(_E, 1, _NC))
    return logits, counts, imp, aux


def kernel(x, params):
    h_feat = _run_encoder(x, params)
    logits, counts, imp, aux = _run_head(h_feat, params)
    return (logits, jnp.reshape(aux, ()), counts.reshape(_E),
            imp.reshape(_E))


# speedup vs baseline: 1.9574x; 1.9574x over previous
import jax
import jax.numpy as jnp
from jax.experimental import pallas as pl
from jax.experimental.pallas import tpu as pltpu

_DIM = 512
_E = 8
_NC = 1000
_B = 32


def _pack_conv_w(w, P):
    Cout, Cin, KH, KW = w.shape
    S = P + 2
    w_t = jnp.transpose(w, (2, 3, 1, 0))
    cols = []
    for p in range(P):
        blk = jnp.zeros((KH, S, Cin, Cout), w.dtype)
        blk = blk.at[:, p:p + 3].set(w_t)
        cols.append(blk)
    Wp = jnp.stack(cols, axis=3)
    return Wp.reshape(KH * S * Cin, P * Cout)


def _pack_conv1_w(w, P):
    Cout, Cin, KH, KW = w.shape
    S = P + 2
    w_t = jnp.transpose(w, (1, 2, 3, 0))
    cols = []
    for p in range(P):
        blk = jnp.zeros((Cin, KH, S, Cout), w.dtype)
        blk = blk.at[:, :, p:p + 3].set(w_t)
        cols.append(blk)
    Wp = jnp.stack(cols, axis=3)
    return Wp.reshape(Cin * KH * S, P * Cout)


def _ktile(a, CL):
    H = a.shape[0] - 2
    G = a.shape[1]
    L = a.shape[2]
    z = jnp.zeros((H, 1, CL), a.dtype)
    parts = []
    for dh in range(3):
        m = a[dh:dh + H]
        left = jnp.concatenate([z, m[:, :G - 1, L - CL:]], axis=1)
        right = jnp.concatenate([m[:, 1:, :CL], z], axis=1)
        parts += [left, m, right]
    return jnp.concatenate(parts, axis=-1)


def _enc_kernel(xg_ref, w1_ref, b1_ref, w2_ref, b2_ref, w3_ref, b3_ref,
                w4_ref, b4_ref, out_ref):
    f32 = jnp.float32
    xg = xg_ref[0]
    k1 = jnp.concatenate(
        [xg[c, dh:dh + 224] for c in range(3) for dh in range(3)], axis=-1)
    a1 = jnp.dot(k1.reshape(6272, 90), w1_ref[...],
                 preferred_element_type=f32)
    a1 = jax.nn.relu(a1 + b1_ref[...])
    act1 = a1.reshape(224, 28, 256).astype(jnp.bfloat16)
    act1 = jnp.pad(act1, ((1, 1), (0, 0), (0, 0)))
    k2 = _ktile(act1, 32)
    a2 = jnp.dot(k2.reshape(6272, 960), w2_ref[...],
                 preferred_element_type=f32)
    a2 = jax.nn.relu(a2 + b2_ref[...]).reshape(112, 2, 28, 256)
    s = a2[:, 0] + a2[:, 1]
    s4 = jnp.concatenate(
        [s[..., 0:32] + s[..., 32:64], s[..., 64:96] + s[..., 96:128],
         s[..., 128:160] + s[..., 160:192],
         s[..., 192:224] + s[..., 224:256]], axis=-1)
    act2 = (s4 * 0.25).astype(jnp.bfloat16)
    act2 = jnp.pad(act2, ((1, 1), (0, 0), (0, 0)))
    k3 = _ktile(act2, 32)
    a3 = jnp.dot(k3.reshape(3136, 576), w3_ref[...],
                 preferred_element_type=f32)
    a3 = jax.nn.relu(a3 + b3_ref[...]).reshape(56, 2, 28, 256)
    s = a3[:, 0] + a3[:, 1]
    s4 = jnp.concatenate(
        [s[..., 0:64] + s[..., 64:128], s[..., 128:192] + s[..., 192:256]],
        axis=-1)
    act3 = (s4.reshape(56, 14, 256) * 0.25).astype(jnp.bfloat16)
    act3 = jnp.pad(act3, ((1, 1), (0, 0), (0, 0)))
    k4 = _ktile(act3, 64)
    a4 = jnp.dot(k4.reshape(784, 1152), w4_ref[...],
                 preferred_element_type=f32)
    a4 = jax.nn.relu(a4 + b4_ref[...])
    r = jnp.sum(a4, axis=0, keepdims=True)
    hsum = (r[:, 0:64] + r[:, 64:128] + r[:, 128:192] + r[:, 192:256])
    out_ref[0] = hsum * (1.0 / 3136.0)


def _run_encoder(x, p):
    xb = x.astype(jnp.bfloat16)
    xp = jnp.pad(xb, ((0, 0), (0, 0), (0, 0), (1, 7)))
    g = xp.reshape(32, 3, 224, 29, 8)
    main = g[:, :, :, 0:28, :]
    extra = g[:, :, :, 1:29, 0:2]
    xg = jnp.concatenate([main, extra], axis=-1)
    xg = jnp.pad(xg, ((0, 0), (0, 0), (1, 1), (0, 0), (0, 0)))

    bf = jnp.bfloat16
    w1 = _pack_conv1_w(p['c1w'], 8).astype(bf)
    w2 = _pack_conv_w(p['c2w'], 8).astype(bf)
    w3 = _pack_conv_w(p['c3w'], 4).astype(bf)
    w4 = _pack_conv_w(p['c4w'], 4).astype(bf)
    b1 = jnp.tile(p['c1b'], 8).reshape(1, 256)
    b2 = jnp.tile(p['c2b'], 8).reshape(1, 256)
    b3 = jnp.tile(p['c3b'], 4).reshape(1, 256)
    b4 = jnp.tile(p['c4b'], 4).reshape(1, 256)

    full = lambda shp: pl.BlockSpec(shp, lambda n: (0,) * len(shp))
    h_feat = pl.pallas_call(
        _enc_kernel,
        grid=(32,),
        in_specs=[
            pl.BlockSpec((1, 3, 226, 28, 10), lambda n: (n, 0, 0, 0, 0)),
            full((90, 256)), full((1, 256)),
            full((960, 256)), full((1, 256)),
            full((576, 256)), full((1, 256)),
            full((1152, 256)), full((1, 256)),
        ],
        out_specs=pl.BlockSpec((1, 1, 64), lambda n: (n, 0, 0)),
        out_shape=jax.ShapeDtypeStruct((32, 1, 64), jnp.float32),
    )(xg, w1, b1, w2, b2, w3, b3, w4, b4)
    return h_feat.reshape(32, 64)


def _head_kernel(h_ref, pw_ref, pb_ref, rw_ref, rb_ref,
                 ew1_ref, eb1_ref, ew2_ref, eb2_ref,
                 logits_ref, counts_ref, imp_ref, aux_ref,
                 z_s, oh_s):
    e = pl.program_id(0)
    f32 = jnp.float32
    bf = jnp.bfloat16

    @pl.when(e == 0)
    def _router():
        h = h_ref[...]
        z = jnp.dot(h.astype(bf), pw_ref[...].astype(bf),
                    preferred_element_type=f32) + pb_ref[...]
        gl = jnp.dot(z.astype(bf), rw_ref[...].astype(bf),
                     preferred_element_type=f32) + rb_ref[...]
        probs = jax.nn.softmax(gl, axis=-1)
        best = probs[:, 0:1]
        idx = jnp.zeros((32, 1), jnp.int32)
        for j in range(1, 8):
            cur = probs[:, j:j + 1]
            upd = cur > best
            best = jnp.where(upd, cur, best)
            idx = jnp.where(upd, jnp.int32(j), idx)
        lane = jax.lax.broadcasted_iota(jnp.int32, (32, 8), 1)
        onehot = (lane == idx).astype(f32)
        counts = jnp.sum(onehot, axis=0, keepdims=True) * (1.0 / 32.0)
        imp = jnp.sum(probs, axis=0, keepdims=True) * (1.0 / 32.0)
        counts_ref[...] = counts
        imp_ref[...] = imp
        aux_ref[...] = 8.0 * jnp.sum(counts * imp).reshape(1, 1)
        z_s[...] = z
        oh_s[...] = onehot

    z = z_s[...]
    h1 = jnp.dot(z.astype(bf), ew1_ref[0].astype(bf),
                 preferred_element_type=f32) + eb1_ref[0]
    h1 = jax.nn.relu(h1)
    o = jnp.dot(h1.astype(bf), ew2_ref[0].astype(bf),
                preferred_element_type=f32) + eb2_ref[0]
    m = jnp.sum(oh_s[...] * (jax.lax.broadcasted_iota(
        jnp.int32, (32, 8), 1) == e).astype(f32), axis=1, keepdims=True)
    val = o * m

    @pl.when(e == 0)
    def _init():
        logits_ref[...] = val

    @pl.when(e != 0)
    def _acc():
        logits_ref[...] = logits_ref[...] + val


def _run_head(h_feat, p):
    full = lambda shp: pl.BlockSpec(shp, lambda e: (0,) * len(shp))
    out_shapes = [
        jax.ShapeDtypeStruct((32, _NC), jnp.float32),
        jax.ShapeDtypeStruct((1, _E), jnp.float32),
        jax.ShapeDtypeStruct((1, _E), jnp.float32),
        jax.ShapeDtypeStruct((1, 1), jnp.float32),
    ]
    logits, counts, imp, aux = pl.pallas_call(
        _head_kernel,
        grid=(_E,),
        in_specs=[
            full((32, 64)),
            full((64, _DIM)), full((1, _DIM)),
            full((_DIM, _E)), full((1, _E)),
            pl.BlockSpec((1, _DIM, _DIM), lambda e: (e, 0, 0)),
            pl.BlockSpec((1, 1, _DIM), lambda e: (e, 0, 0)),
            pl.BlockSpec((1, _DIM, _NC), lambda e: (e, 0, 0)),
            pl.BlockSpec((1, 1, _NC), lambda e: (e, 0, 0)),
        ],
        out_specs=[
            full((32, _NC)), full((1, _E)), full((1, _E)), full((1, 1)),
        ],
        out_shape=out_shapes,
        scratch_shapes=[
            pltpu.VMEM((32, _DIM), jnp.float32),
            pltpu.VMEM((32, _E), jnp.float32),
        ],
    )(h_feat, p['pw'], p['pb'].reshape(1, _DIM), p['rw'],
      p['rb'].reshape(1, _E), p['ew1'], p['eb1'].reshape(_E, 1, _DIM),
      p['ew2'], p['eb2'].reshape(_E, 1, _NC))
    return logits, counts, imp, aux


def kernel(x, params):
    h_feat = _run_encoder(x, params)
    logits, counts, imp, aux = _run_head(h_feat, params)
    return (logits, jnp.reshape(aux, ()), counts.reshape(_E),
            imp.reshape(_E))


# speedup vs baseline: 2.2688x; 1.1591x over previous
import jax
import jax.numpy as jnp
from jax.experimental import pallas as pl
from jax.experimental.pallas import tpu as pltpu

_DIM = 512
_E = 8
_NC = 1000
_B = 32


def _pack_conv_w(w, P):
    Cout, Cin, KH, KW = w.shape
    S = P + 2
    w_t = jnp.transpose(w, (2, 3, 1, 0))
    cols = []
    for p in range(P):
        blk = jnp.zeros((KH, S, Cin, Cout), w.dtype)
        blk = blk.at[:, p:p + 3].set(w_t)
        cols.append(blk)
    Wp = jnp.stack(cols, axis=3)
    perm = list(range(1, P + 1)) + [0, P + 1]
    Wp = Wp[:, jnp.array(perm)]
    return Wp.reshape(KH, S * Cin, P * Cout)


def _pack_conv1_w(w, P):
    Cout, Cin, KH, KW = w.shape
    S = P + 2
    w_t = jnp.transpose(w, (2, 1, 3, 0))
    cols = []
    for p in range(P):
        blk = jnp.zeros((KH, Cin, S, Cout), w.dtype)
        blk = blk.at[:, :, p:p + 3].set(w_t)
        cols.append(blk)
    Wp = jnp.stack(cols, axis=3)
    return Wp.reshape(KH, Cin * S, P * Cout)


def _enc_kernel(xg_ref, w1_ref, b1_ref, w2_ref, b2_ref, w3_ref, b3_ref,
                w4_ref, b4_ref, out_ref, s2, s3, s4):
    f32 = jnp.float32
    bf = jnp.bfloat16

    @pl.when(pl.program_id(0) == 0)
    def _init_edges():
        s2[...] = jnp.zeros((226, 28, 320), bf)
        s3[...] = jnp.zeros((114, 28, 192), bf)
        s4[...] = jnp.zeros((58, 14, 384), bf)

    a1 = jnp.dot(xg_ref[0, 0:224].reshape(6272, 30), w1_ref[0],
                 preferred_element_type=f32)
    a1 += jnp.dot(xg_ref[0, 1:225].reshape(6272, 30), w1_ref[1],
                  preferred_element_type=f32)
    a1 += jnp.dot(xg_ref[0, 2:226].reshape(6272, 30), w1_ref[2],
                  preferred_element_type=f32)
    act1 = jax.nn.relu(a1 + b1_ref[...]).astype(bf).reshape(224, 28, 256)
    s2[1:225, :, 0:256] = act1
    s2[1:225, 1:28, 256:288] = act1[:, 0:27, 224:256]
    s2[1:225, 0:27, 288:320] = act1[:, 1:28, 0:32]

    a2 = jnp.dot(s2[0:224].reshape(6272, 320), w2_ref[0],
                 preferred_element_type=f32)
    a2 += jnp.dot(s2[1:225].reshape(6272, 320), w2_ref[1],
                  preferred_element_type=f32)
    a2 += jnp.dot(s2[2:226].reshape(6272, 320), w2_ref[2],
                  preferred_element_type=f32)
    a2 = jax.nn.relu(a2 + b2_ref[...]).reshape(112, 2, 28, 256)
    s = a2[:, 0] + a2[:, 1]
    ps = jnp.concatenate(
        [s[..., 0:32] + s[..., 32:64], s[..., 64:96] + s[..., 96:128],
         s[..., 128:160] + s[..., 160:192],
         s[..., 192:224] + s[..., 224:256]], axis=-1)
    act2 = (ps * 0.25).astype(bf)
    s3[1:113, :, 0:128] = act2
    s3[1:113, 1:28, 128:160] = act2[:, 0:27, 96:128]
    s3[1:113, 0:27, 160:192] = act2[:, 1:28, 0:32]

    a3 = jnp.dot(s3[0:112].reshape(3136, 192), w3_ref[0],
                 preferred_element_type=f32)
    a3 += jnp.dot(s3[1:113].reshape(3136, 192), w3_ref[1],
                  preferred_element_type=f32)
    a3 += jnp.dot(s3[2:114].reshape(3136, 192), w3_ref[2],
                  preferred_element_type=f32)
    a3 = jax.nn.relu(a3 + b3_ref[...]).reshape(56, 2, 28, 256)
    s = a3[:, 0] + a3[:, 1]
    ps = jnp.concatenate(
        [s[..., 0:64] + s[..., 64:128], s[..., 128:192] + s[..., 192:256]],
        axis=-1)
    act3 = (ps.reshape(56, 14, 256) * 0.25).astype(bf)
    s4[1:57, :, 0:256] = act3
    s4[1:57, 1:14, 256:320] = act3[:, 0:13, 192:256]
    s4[1:57, 0:13, 320:384] = act3[:, 1:14, 0:64]

    a4 = jnp.dot(s4[0:56].reshape(784, 384), w4_ref[0],
                 preferred_element_type=f32)
    a4 += jnp.dot(s4[1:57].reshape(784, 384), w4_ref[1],
                  preferred_element_type=f32)
    a4 += jnp.dot(s4[2:58].reshape(784, 384), w4_ref[2],
                  preferred_element_type=f32)
    a4 = jax.nn.relu(a4 + b4_ref[...])
    r = jnp.sum(a4, axis=0, keepdims=True)
    hsum = (r[:, 0:64] + r[:, 64:128] + r[:, 128:192] + r[:, 192:256])
    out_ref[0] = hsum * (1.0 / 3136.0)


def _run_encoder(x, p):
    xb = x.astype(jnp.bfloat16)
    xp = jnp.pad(xb, ((0, 0), (0, 0), (0, 0), (1, 7)))
    g = xp.reshape(32, 3, 224, 29, 8)
    main = g[:, :, :, 0:28, :]
    extra = g[:, :, :, 1:29, 0:2]
    xgc = jnp.concatenate([main, extra], axis=-1)
    xgc = jnp.pad(xgc, ((0, 0), (0, 0), (1, 1), (0, 0), (0, 0)))
    xg = jnp.concatenate([xgc[:, 0], xgc[:, 1], xgc[:, 2]], axis=-1)

    bf = jnp.bfloat16
    w1 = _pack_conv1_w(p['c1w'], 8).astype(bf)
    w2 = _pack_conv_w(p['c2w'], 8).astype(bf)
    w3 = _pack_conv_w(p['c3w'], 4).astype(bf)
    w4 = _pack_conv_w(p['c4w'], 4).astype(bf)
    b1 = jnp.tile(p['c1b'], 8).reshape(1, 256)
    b2 = jnp.tile(p['c2b'], 8).reshape(1, 256)
    b3 = jnp.tile(p['c3b'], 4).reshape(1, 256)
    b4 = jnp.tile(p['c4b'], 4).reshape(1, 256)

    full = lambda shp: pl.BlockSpec(shp, lambda n: (0,) * len(shp))
    h_feat = pl.pallas_call(
        _enc_kernel,
        grid=(32,),
        in_specs=[
            pl.BlockSpec((1, 226, 28, 30), lambda n: (n, 0, 0, 0)),
            full((3, 30, 256)), full((1, 256)),
            full((3, 320, 256)), full((1, 256)),
            full((3, 192, 256)), full((1, 256)),
            full((3, 384, 256)), full((1, 256)),
        ],
        out_specs=pl.BlockSpec((1, 1, 64), lambda n: (n, 0, 0)),
        out_shape=jax.ShapeDtypeStruct((32, 1, 64), jnp.float32),
        scratch_shapes=[
            pltpu.VMEM((226, 28, 320), jnp.bfloat16),
            pltpu.VMEM((114, 28, 192), jnp.bfloat16),
            pltpu.VMEM((58, 14, 384), jnp.bfloat16),
        ],
    )(xg, w1, b1, w2, b2, w3, b3, w4, b4)
    return h_feat.reshape(32, 64)


def _head_kernel(h_ref, pw_ref, pb_ref, rw_ref, rb_ref,
                 ew1_ref, eb1_ref, ew2_ref, eb2_ref,
                 logits_ref, counts_ref, imp_ref, aux_ref,
                 z_s, oh_s):
    e = pl.program_id(0)
    f32 = jnp.float32
    bf = jnp.bfloat16

    @pl.when(e == 0)
    def _router():
        h = h_ref[...]
        z = jnp.dot(h.astype(bf), pw_ref[...].astype(bf),
                    preferred_element_type=f32) + pb_ref[...]
        gl = jnp.dot(z.astype(bf), rw_ref[...].astype(bf),
                     preferred_element_type=f32) + rb_ref[...]
        probs = jax.nn.softmax(gl, axis=-1)
        best = probs[:, 0:1]
        idx = jnp.zeros((32, 1), jnp.int32)
        for j in range(1, 8):
            cur = probs[:, j:j + 1]
            upd = cur > best
            best = jnp.where(upd, cur, best)
            idx = jnp.where(upd, jnp.int32(j), idx)
        lane = jax.lax.broadcasted_iota(jnp.int32, (32, 8), 1)
        onehot = (lane == idx).astype(f32)
        counts = jnp.sum(onehot, axis=0, keepdims=True) * (1.0 / 32.0)
        imp = jnp.sum(probs, axis=0, keepdims=True) * (1.0 / 32.0)
        counts_ref[...] = counts
        imp_ref[...] = imp
        aux_ref[...] = 8.0 * jnp.sum(counts * imp).reshape(1, 1)
        z_s[...] = z
        oh_s[...] = onehot

    z = z_s[...]
    h1 = jnp.dot(z.astype(bf), ew1_ref[0].astype(bf),
                 preferred_element_type=f32) + eb1_ref[0]
    h1 = jax.nn.relu(h1)
    o = jnp.dot(h1.astype(bf), ew2_ref[0].astype(bf),
                preferred_element_type=f32) + eb2_ref[0]
    m = jnp.sum(oh_s[...] * (jax.lax.broadcasted_iota(
        jnp.int32, (32, 8), 1) == e).astype(f32), axis=1, keepdims=True)
    val = o * m

    @pl.when(e == 0)
    def _init():
        logits_ref[...] = val

    @pl.when(e != 0)
    def _acc():
        logits_ref[...] = logits_ref[...] + val


def _run_head(h_feat, p):
    full = lambda shp: pl.BlockSpec(shp, lambda e: (0,) * len(shp))
    out_shapes = [
        jax.ShapeDtypeStruct((32, _NC), jnp.float32),
        jax.ShapeDtypeStruct((1, _E), jnp.float32),
        jax.ShapeDtypeStruct((1, _E), jnp.float32),
        jax.ShapeDtypeStruct((1, 1), jnp.float32),
    ]
    logits, counts, imp, aux = pl.pallas_call(
        _head_kernel,
        grid=(_E,),
        in_specs=[
            full((32, 64)),
            full((64, _DIM)), full((1, _DIM)),
            full((_DIM, _E)), full((1, _E)),
            pl.BlockSpec((1, _DIM, _DIM), lambda e: (e, 0, 0)),
            pl.BlockSpec((1, 1, _DIM), lambda e: (e, 0, 0)),
            pl.BlockSpec((1, _DIM, _NC), lambda e: (e, 0, 0)),
            pl.BlockSpec((1, 1, _NC), lambda e: (e, 0, 0)),
        ],
        out_specs=[
            full((32, _NC)), full((1, _E)), full((1, _E)), full((1, 1)),
        ],
        out_shape=out_shapes,
        scratch_shapes=[
            pltpu.VMEM((32, _DIM), jnp.float32),
            pltpu.VMEM((32, _E), jnp.float32),
        ],
    )(h_feat, p['pw'], p['pb'].reshape(1, _DIM), p['rw'],
      p['rb'].reshape(1, _E), p['ew1'], p['eb1'].reshape(_E, 1, _DIM),
      p['ew2'], p['eb2'].reshape(_E, 1, _NC))
    return logits, counts, imp, aux


def kernel(x, params):
    h_feat = _run_encoder(x, params)
    logits, counts, imp, aux = _run_head(h_feat, params)
    return (logits, jnp.reshape(aux, ()), counts.reshape(_E),
            imp.reshape(_E))
